# Initial kernel scaffold; baseline (speedup 1.0000x reference)
import jax
import jax.numpy as jnp
from jax.experimental import pallas as pl


def kernel(x, edge_index, edge_type, rel_emb, W_e2r):
    raise NotImplementedError("write your pallas kernel here")


# speedup vs baseline: 23.2759x; 23.2759x over previous
import functools

import jax
import jax.numpy as jnp
from jax import lax
from jax.experimental import pallas as pl
from jax.experimental.pallas import tpu as pltpu
from jax.experimental.pallas import tpu_sc as plsc

N_NODES = 10000
N_EDGES = 320000
HID = 128
NUM_REL = 500
RPAD = 512

NC, NS = 2, 16
NW = NC * NS
E_PER = N_EDGES // NW
CH = 128
NCH = (E_PER + CH - 1) // CH
NROW = 80
NVEC = E_PER // 16
MB = 2000


def _mm_body(x_ref, relp_ref, w_ref, o_ref):
    e_x = lax.dot_general(
        x_ref[...], w_ref[...], (((1,), (1,)), ((), ())),
        preferred_element_type=jnp.float32)
    o_ref[...] = lax.dot_general(
        e_x, relp_ref[...], (((1,), (1,)), ((), ())),
        precision=lax.Precision.HIGHEST, preferred_element_type=jnp.float32)


def _matmul(x, relp, w):
    return pl.pallas_call(
        _mm_body,
        grid=(N_NODES // MB,),
        in_specs=[
            pl.BlockSpec((MB, HID), lambda i: (i, 0)),
            pl.BlockSpec((RPAD, HID), lambda i: (0, 0)),
            pl.BlockSpec((HID, HID), lambda i: (0, 0)),
        ],
        out_specs=pl.BlockSpec((MB, RPAD), lambda i: (i, 0)),
        out_shape=jax.ShapeDtypeStruct((N_NODES, RPAD), jnp.float32),
    )(x, relp, w)


def _sc_body(mflat, head_hbm, type_hbm, hist_hbm, maxes_hbm,
             head_v, type_v, idx_v, dp_v, bins, row_v, max_v, sem):
    wid = lax.axis_index("s") * NC + lax.axis_index("c")
    base = wid * E_PER
    pltpu.sync_copy(head_hbm.at[pl.ds(base, E_PER)], head_v)
    pltpu.sync_copy(type_hbm.at[pl.ds(base, E_PER)], type_v)

    zeros16 = jnp.zeros((16,), jnp.float32)

    def zero_body(i, _):
        bins[i >> 5, pl.ds((i & 31) * 16, 16)] = zeros16
        return 0
    lax.fori_loop(0, 16 * (RPAD // 16), zero_body, 0)

    def idx_body(i, _):
        h = head_v[pl.ds(i * 16, 16)]
        t = type_v[pl.ds(i * 16, 16)]
        idx_v[i >> 3, pl.ds((i & 7) * 16, 16)] = h * RPAD + t
        return 0
    lax.fori_loop(0, NVEC, idx_body, 0)

    zeros16i = jnp.zeros((16,), jnp.int32)

    def pad_body(i, _):
        idx_v[i >> 3, pl.ds((i & 7) * 16, 16)] = zeros16i
        return 0
    lax.fori_loop(NVEC, NROW * 8, pad_body, 0)

    def fire(j, _):
        pltpu.async_copy(mflat.at[idx_v.at[j]], dp_v.at[j], sem)
        return 0
    lax.fori_loop(0, NROW, fire, 0)

    def drain(j, _):
        pltpu.make_async_copy(mflat.at[idx_v.at[j]], dp_v.at[j], sem).wait()
        return 0
    lax.fori_loop(0, NROW, drain, 0)

    def max_body(i, acc):
        return jnp.maximum(acc, dp_v[i >> 3, pl.ds((i & 7) * 16, 16)])
    mvec = lax.fori_loop(0, NVEC, max_body,
                         jnp.full((16,), -jnp.inf, jnp.float32))

    lane = lax.iota(jnp.int32, 16)

    for sh in (8, 4, 2, 1):
        max_v[...] = mvec
        mvec = jnp.maximum(mvec, plsc.load_gather(max_v, [lane ^ sh]))
    mv = mvec

    def exp_body(i, _):
        v = dp_v[i >> 3, pl.ds((i & 7) * 16, 16)]
        t = type_v[pl.ds(i * 16, 16)]
        plsc.addupdate_scatter(bins, [lane, t], jnp.exp(v - mv))
        return 0
    lax.fori_loop(0, NVEC, exp_body, 0)

    def merge_body(c, _):
        acc = bins[0, pl.ds(c * 16, 16)]
        for k in range(1, 16):
            acc = acc + bins[k, pl.ds(c * 16, 16)]
        row_v[pl.ds(c * 16, 16)] = acc
        return 0
    lax.fori_loop(0, RPAD // 16, merge_body, 0)

    max_v[...] = mv
    pltpu.sync_copy(row_v, hist_hbm.at[wid])
    pltpu.sync_copy(max_v, maxes_hbm.at[wid])


_sc_edge = pl.kernel(
    _sc_body,
    out_type=(jax.ShapeDtypeStruct((NW, RPAD), jnp.float32),
              jax.ShapeDtypeStruct((NW, 16), jnp.float32)),
    mesh=plsc.VectorSubcoreMesh(core_axis_name="c", subcore_axis_name="s"),
    compiler_params=pltpu.CompilerParams(needs_layout_passes=False),
    scratch_types=[
        pltpu.VMEM((E_PER,), jnp.int32),
        pltpu.VMEM((E_PER,), jnp.int32),
        pltpu.VMEM((NROW, CH), jnp.int32),
        pltpu.VMEM((NROW, CH), jnp.float32),
        pltpu.VMEM((16, RPAD), jnp.float32),
        pltpu.VMEM((RPAD,), jnp.float32),
        pltpu.VMEM((16,), jnp.float32),
        pltpu.SemaphoreType.DMA,
    ],
)


def _fin_body(hist_ref, maxes_ref, rel_ref, o_ref):
    m = jnp.max(maxes_ref[...])
    e = jnp.exp(maxes_ref[:, 0:1] - m)
    t = jnp.sum(hist_ref[...] * e, axis=0)
    z = jnp.sum(t)
    s = t[0:NUM_REL] / z
    o_ref[...] = jnp.maximum(rel_ref[...] * s[:, None], 0.0)


def _finalize(hist, maxes, rel_emb):
    return pl.pallas_call(
        _fin_body,
        out_shape=jax.ShapeDtypeStruct((NUM_REL, HID), jnp.float32),
    )(hist, maxes, rel_emb)


def kernel(x, edge_index, edge_type, rel_emb, W_e2r):
    head = edge_index[0]
    relp = jnp.pad(rel_emb, ((0, RPAD - NUM_REL), (0, 0)))
    m = _matmul(x, relp, W_e2r)
    hist, maxes = _sc_edge(m.reshape(-1), head, edge_type)
    return _finalize(hist, maxes, rel_emb)


# speedup vs baseline: 24.5411x; 1.0544x over previous
import functools

import jax
import jax.numpy as jnp
from jax import lax
from jax.experimental import pallas as pl
from jax.experimental.pallas import tpu as pltpu
from jax.experimental.pallas import tpu_sc as plsc

N_NODES = 10000
N_EDGES = 320000
HID = 128
NUM_REL = 500
RPAD = 512

NC, NS = 2, 16
NW = NC * NS
E_PER = N_EDGES // NW
CH = 128
NCH = (E_PER + CH - 1) // CH
NROW = 80
NVEC = E_PER // 16
MB = 2000


def _mm_body(x_ref, relp_ref, w_ref, o_ref):
    e_x = lax.dot_general(
        x_ref[...], w_ref[...], (((1,), (1,)), ((), ())),
        preferred_element_type=jnp.float32)
    o_ref[...] = lax.dot_general(
        e_x, relp_ref[...], (((1,), (1,)), ((), ())),
        precision=lax.Precision.HIGHEST, preferred_element_type=jnp.float32)


def _matmul(x, relp, w):
    return pl.pallas_call(
        _mm_body,
        grid=(N_NODES // MB,),
        in_specs=[
            pl.BlockSpec((MB, HID), lambda i: (i, 0)),
            pl.BlockSpec((RPAD, HID), lambda i: (0, 0)),
            pl.BlockSpec((HID, HID), lambda i: (0, 0)),
        ],
        out_specs=pl.BlockSpec((MB, RPAD), lambda i: (i, 0)),
        out_shape=jax.ShapeDtypeStruct((N_NODES, RPAD), jnp.float32),
    )(x, relp, w)


def _sc_body(mflat, head_hbm, type_hbm, hist_hbm, maxes_hbm,
             head_v, type_v, idx_v, dp_v, bins, row_v, max_v, sem):
    wid = lax.axis_index("s") * NC + lax.axis_index("c")
    base = wid * E_PER
    pltpu.sync_copy(head_hbm.at[pl.ds(base, E_PER)], head_v)
    pltpu.sync_copy(type_hbm.at[pl.ds(base, E_PER)], type_v)

    zeros16 = jnp.zeros((16,), jnp.float32)
    zeros16i = jnp.zeros((16,), jnp.int32)
    NFULL = NVEC // 8

    def zero_row(k, _):
        for c in range(RPAD // 16):
            bins[k, pl.ds(c * 16, 16)] = zeros16
        return 0
    lax.fori_loop(0, 16, zero_row, 0)

    def row_body(j, _):
        for k in range(8):
            h = head_v[pl.ds(j * CH + k * 16, 16)]
            t = type_v[pl.ds(j * CH + k * 16, 16)]
            idx_v[j, pl.ds(k * 16, 16)] = h * RPAD + t
        pltpu.async_copy(mflat.at[idx_v.at[j]], dp_v.at[j], sem)
        return 0
    lax.fori_loop(0, NFULL, row_body, 0)

    h = head_v[pl.ds(NFULL * CH, 16)]
    t = type_v[pl.ds(NFULL * CH, 16)]
    idx_v[NFULL, pl.ds(0, 16)] = h * RPAD + t
    for k in range(1, 8):
        idx_v[NFULL, pl.ds(k * 16, 16)] = zeros16i
        idx_v[NFULL + 1, pl.ds(k * 16, 16)] = zeros16i
    idx_v[NFULL + 1, pl.ds(0, 16)] = zeros16i
    pltpu.async_copy(mflat.at[idx_v.at[NFULL]], dp_v.at[NFULL], sem)
    pltpu.async_copy(mflat.at[idx_v.at[NFULL + 1]], dp_v.at[NFULL + 1], sem)

    def drain(j, _):
        pltpu.make_async_copy(mflat.at[idx_v.at[j]], dp_v.at[j], sem).wait()
        return 0
    lax.fori_loop(0, NROW, drain, 0)

    def max_row(j, acc):
        for k in range(8):
            acc = jnp.maximum(acc, dp_v[j, pl.ds(k * 16, 16)])
        return acc
    mvec = lax.fori_loop(0, NFULL, max_row,
                         jnp.full((16,), -jnp.inf, jnp.float32))
    mvec = jnp.maximum(mvec, dp_v[NFULL, pl.ds(0, 16)])

    lane = lax.iota(jnp.int32, 16)

    for sh in (8, 4, 2, 1):
        max_v[...] = mvec
        mvec = jnp.maximum(mvec, plsc.load_gather(max_v, [lane ^ sh]))
    mv = mvec

    def exp_row(j, _):
        for k in range(8):
            v = dp_v[j, pl.ds(k * 16, 16)]
            t = type_v[pl.ds(j * CH + k * 16, 16)]
            plsc.addupdate_scatter(bins, [lane, t], jnp.exp(v - mv))
        return 0
    lax.fori_loop(0, NFULL, exp_row, 0)
    v = dp_v[NFULL, pl.ds(0, 16)]
    t = type_v[pl.ds(NFULL * CH, 16)]
    plsc.addupdate_scatter(bins, [lane, t], jnp.exp(v - mv))

    def merge_body(c, _):
        acc = bins[0, pl.ds(c * 16, 16)]
        for k in range(1, 16):
            acc = acc + bins[k, pl.ds(c * 16, 16)]
        row_v[pl.ds(c * 16, 16)] = acc
        return 0
    lax.fori_loop(0, RPAD // 16, merge_body, 0)

    max_v[...] = mv
    pltpu.sync_copy(row_v, hist_hbm.at[wid])
    pltpu.sync_copy(max_v, maxes_hbm.at[wid])


_sc_edge = pl.kernel(
    _sc_body,
    out_type=(jax.ShapeDtypeStruct((NW, RPAD), jnp.float32),
              jax.ShapeDtypeStruct((NW, 16), jnp.float32)),
    mesh=plsc.VectorSubcoreMesh(core_axis_name="c", subcore_axis_name="s"),
    compiler_params=pltpu.CompilerParams(needs_layout_passes=False),
    scratch_types=[
        pltpu.VMEM((E_PER,), jnp.int32),
        pltpu.VMEM((E_PER,), jnp.int32),
        pltpu.VMEM((NROW, CH), jnp.int32),
        pltpu.VMEM((NROW, CH), jnp.float32),
        pltpu.VMEM((16, RPAD), jnp.float32),
        pltpu.VMEM((RPAD,), jnp.float32),
        pltpu.VMEM((16,), jnp.float32),
        pltpu.SemaphoreType.DMA,
    ],
)


def _fin_body(hist_ref, maxes_ref, rel_ref, o_ref):
    m = jnp.max(maxes_ref[...])
    e = jnp.exp(maxes_ref[:, 0:1] - m)
    t = jnp.sum(hist_ref[...] * e, axis=0)
    z = jnp.sum(t)
    s = t[0:NUM_REL] / z
    o_ref[...] = jnp.maximum(rel_ref[...] * s[:, None], 0.0)


def _finalize(hist, maxes, rel_emb):
    return pl.pallas_call(
        _fin_body,
        out_shape=jax.ShapeDtypeStruct((NUM_REL, HID), jnp.float32),
    )(hist, maxes, rel_emb)


def kernel(x, edge_index, edge_type, rel_emb, W_e2r):
    head = edge_index[0]
    relp = jnp.pad(rel_emb, ((0, RPAD - NUM_REL), (0, 0)))
    m = _matmul(x, relp, W_e2r)
    hist, maxes = _sc_edge(m.reshape(-1), head, edge_type)
    return _finalize(hist, maxes, rel_emb)


# speedup vs baseline: 37.1910x; 1.5155x over previous
import functools

import jax
import jax.numpy as jnp
from jax import lax
from jax.experimental import pallas as pl
from jax.experimental.pallas import tpu as pltpu
from jax.experimental.pallas import tpu_sc as plsc

N_NODES = 10000
N_EDGES = 320000
HID = 128
NUM_REL = 500
RPAD = 512

NC, NS = 2, 16
NW = NC * NS
E_PER = N_EDGES // NW
CH = 128
NCH = (E_PER + CH - 1) // CH
NROW = 80
NVEC = E_PER // 16
MB = 2000
NPAD = 10240
PLANE = NPAD * HID


def _mm_body(x_ref, relp_ref, w_ref, o_ref):
    e_x = lax.dot_general(
        x_ref[...], w_ref[...], (((1,), (1,)), ((), ())),
        preferred_element_type=jnp.float32)
    relp = relp_ref[...]
    for j in range(4):
        o_ref[j] = lax.dot_general(
            e_x, relp[j * HID:(j + 1) * HID, :], (((1,), (1,)), ((), ())),
            precision=lax.Precision.HIGHEST, preferred_element_type=jnp.float32)


def _matmul(x, relp, w):
    return pl.pallas_call(
        _mm_body,
        grid=(N_NODES // MB,),
        in_specs=[
            pl.BlockSpec((MB, HID), lambda i: (i, 0)),
            pl.BlockSpec((RPAD, HID), lambda i: (0, 0)),
            pl.BlockSpec((HID, HID), lambda i: (0, 0)),
        ],
        out_specs=pl.BlockSpec((4, MB, HID), lambda i: (0, i, 0)),
        out_shape=jax.ShapeDtypeStruct((4, NPAD, HID), jnp.float32),
    )(x, relp, w)


def _sc_body(mflat, head_hbm, type_hbm, hist_hbm, maxes_hbm,
             head_v, type_v, idx_v, dp_v, bins, row_v, max_v, sem):
    wid = lax.axis_index("s") * NC + lax.axis_index("c")
    base = wid * E_PER
    cp1 = pltpu.async_copy(head_hbm.at[pl.ds(base, E_PER)], head_v, sem)
    cp2 = pltpu.async_copy(type_hbm.at[pl.ds(base, E_PER)], type_v, sem)
    cp1.wait()
    cp2.wait()

    zeros16 = jnp.zeros((16,), jnp.float32)
    NFULL = NVEC // 8

    def zero_row(k, _):
        for c in range(RPAD // 16):
            bins[k, pl.ds(c * 16, 16)] = zeros16
        return 0
    lax.fori_loop(0, 16, zero_row, 0)

    lane0 = lax.iota(jnp.int32, 16)

    def _flat_idx(h, t):
        return (t >> 7) * PLANE + (h << 7) + (t & 127)

    def row_body(j, _):
        for k in range(8):
            h = head_v[pl.ds(j * CH + k * 16, 16)]
            t = type_v[pl.ds(j * CH + k * 16, 16)]
            idx_v[j, pl.ds(k * 16, 16)] = _flat_idx(h, t)
        pltpu.async_copy(mflat.at[idx_v.at[j]], dp_v.at[j], sem)
        return 0
    lax.fori_loop(0, NFULL, row_body, 0)

    h = head_v[pl.ds(NFULL * CH, 16)]
    t = type_v[pl.ds(NFULL * CH, 16)]
    idx_v[NFULL, pl.ds(0, 16)] = _flat_idx(h, t)
    pbase = wid * 256
    for k in range(1, 8):
        idx_v[NFULL, pl.ds(k * 16, 16)] = (pbase + (k * 16) + lane0) << 6
        idx_v[NFULL + 1, pl.ds(k * 16, 16)] = (pbase + (128 + k * 16) + lane0) << 6
    idx_v[NFULL + 1, pl.ds(0, 16)] = (pbase + 128 + lane0) << 6
    pltpu.async_copy(mflat.at[idx_v.at[NFULL]], dp_v.at[NFULL], sem)
    pltpu.async_copy(mflat.at[idx_v.at[NFULL + 1]], dp_v.at[NFULL + 1], sem)

    def drain(j, _):
        pltpu.make_async_copy(mflat.at[idx_v.at[j]], dp_v.at[j], sem).wait()
        return 0
    lax.fori_loop(0, NROW, drain, 0)

    def max_row(j, acc):
        for k in range(8):
            acc = jnp.maximum(acc, dp_v[j, pl.ds(k * 16, 16)])
        return acc
    mvec = lax.fori_loop(0, NFULL, max_row,
                         jnp.full((16,), -jnp.inf, jnp.float32))
    mvec = jnp.maximum(mvec, dp_v[NFULL, pl.ds(0, 16)])

    lane = lax.iota(jnp.int32, 16)

    for sh in (8, 4, 2, 1):
        max_v[...] = mvec
        mvec = jnp.maximum(mvec, plsc.load_gather(max_v, [lane ^ sh]))
    mv = mvec

    def exp_row(j, _):
        for k in range(8):
            v = dp_v[j, pl.ds(k * 16, 16)]
            t = type_v[pl.ds(j * CH + k * 16, 16)]
            plsc.addupdate_scatter(bins, [lane, t], jnp.exp(v - mv))
        return 0
    lax.fori_loop(0, NFULL, exp_row, 0)
    v = dp_v[NFULL, pl.ds(0, 16)]
    t = type_v[pl.ds(NFULL * CH, 16)]
    plsc.addupdate_scatter(bins, [lane, t], jnp.exp(v - mv))

    def merge_body(c, _):
        acc = bins[0, pl.ds(c * 16, 16)]
        for k in range(1, 16):
            acc = acc + bins[k, pl.ds(c * 16, 16)]
        row_v[pl.ds(c * 16, 16)] = acc
        return 0
    lax.fori_loop(0, RPAD // 16, merge_body, 0)

    max_v[...] = mv
    pltpu.sync_copy(row_v, hist_hbm.at[wid])
    pltpu.sync_copy(max_v, maxes_hbm.at[wid])


_sc_edge = pl.kernel(
    _sc_body,
    out_type=(jax.ShapeDtypeStruct((NW, RPAD), jnp.float32),
              jax.ShapeDtypeStruct((NW, 16), jnp.float32)),
    mesh=plsc.VectorSubcoreMesh(core_axis_name="c", subcore_axis_name="s"),
    compiler_params=pltpu.CompilerParams(needs_layout_passes=False),
    scratch_types=[
        pltpu.VMEM((E_PER,), jnp.int32),
        pltpu.VMEM((E_PER,), jnp.int32),
        pltpu.VMEM((NROW, CH), jnp.int32),
        pltpu.VMEM((NROW, CH), jnp.float32),
        pltpu.VMEM((16, RPAD), jnp.float32),
        pltpu.VMEM((RPAD,), jnp.float32),
        pltpu.VMEM((16,), jnp.float32),
        pltpu.SemaphoreType.DMA,
    ],
)


def _fin_body(hist_ref, maxes_ref, rel_ref, o_ref):
    m = jnp.max(maxes_ref[...])
    e = jnp.exp(maxes_ref[:, 0:1] - m)
    t = jnp.sum(hist_ref[...] * e, axis=0)
    z = jnp.sum(t)
    s = t[0:NUM_REL] / z
    o_ref[...] = jnp.maximum(rel_ref[...] * s[:, None], 0.0)


def _finalize(hist, maxes, rel_emb):
    return pl.pallas_call(
        _fin_body,
        out_shape=jax.ShapeDtypeStruct((NUM_REL, HID), jnp.float32),
    )(hist, maxes, rel_emb)


def kernel(x, edge_index, edge_type, rel_emb, W_e2r):
    head = edge_index[0]
    relp = jnp.pad(rel_emb, ((0, RPAD - NUM_REL), (0, 0)))
    m = _matmul(x, relp, W_e2r)
    hist, maxes = _sc_edge(m.reshape(-1), head, edge_type)
    return _finalize(hist, maxes, rel_emb)


# speedup vs baseline: 45.0665x; 1.2118x over previous
import functools

import jax
import jax.numpy as jnp
from jax import lax
from jax.experimental import pallas as pl
from jax.experimental.pallas import tpu as pltpu
from jax.experimental.pallas import tpu_sc as plsc

N_NODES = 10000
N_EDGES = 320000
HID = 128
NUM_REL = 500
RPAD = 512

NC, NS = 2, 16
NW = NC * NS
E_PER = N_EDGES // NW
CH = 128
NCH = (E_PER + CH - 1) // CH
NROW = 80
NVEC = E_PER // 16
MB = 2000
NPAD = 10240
PLANE = NPAD * HID


def _mm_body(x_ref, relp_ref, w_ref, o_ref):
    e_x = lax.dot_general(
        x_ref[...], w_ref[...], (((1,), (1,)), ((), ())),
        preferred_element_type=jnp.float32)
    relp = relp_ref[...]
    for j in range(4):
        o_ref[j] = lax.dot_general(
            e_x, relp[j * HID:(j + 1) * HID, :], (((1,), (1,)), ((), ())),
            precision=lax.Precision.HIGHEST, preferred_element_type=jnp.float32)


def _matmul(x, relp, w):
    return pl.pallas_call(
        _mm_body,
        grid=(N_NODES // MB,),
        in_specs=[
            pl.BlockSpec((MB, HID), lambda i: (i, 0)),
            pl.BlockSpec((RPAD, HID), lambda i: (0, 0)),
            pl.BlockSpec((HID, HID), lambda i: (0, 0)),
        ],
        out_specs=pl.BlockSpec((4, MB, HID), lambda i: (0, i, 0)),
        out_shape=jax.ShapeDtypeStruct((4, NPAD, HID), jnp.float32),
    )(x, relp, w)


def _sc_body(mflat, eidx_hbm, type_hbm, hist_hbm, maxes_hbm,
             eidx_v, type_v, idx_v, dp_v, bins, row_v, max_v, sem):
    wid = lax.axis_index("s") * NC + lax.axis_index("c")
    base = wid * E_PER
    tile0 = jnp.minimum(base >> 7, jnp.int32(N_EDGES // CH - NROW))
    off0 = base - (tile0 << 7)
    cp1 = pltpu.async_copy(
        eidx_hbm.at[:, pl.ds(pl.multiple_of(tile0 * CH, CH), NROW * CH)],
        eidx_v, sem)
    cp2 = pltpu.async_copy(type_hbm.at[pl.ds(base, E_PER)], type_v, sem)
    cp1.wait()
    cp2.wait()

    zeros16 = jnp.zeros((16,), jnp.float32)
    NFULL = NVEC // 8

    def zero_row(k, _):
        for c in range(RPAD // 16):
            bins[k, pl.ds(c * 16, 16)] = zeros16
        return 0
    lax.fori_loop(0, 16, zero_row, 0)

    lane0 = lax.iota(jnp.int32, 16)

    def _flat_idx(h, t):
        return (t >> 7) * PLANE + (h << 7) + (t & 127)

    def row_body(j, _):
        for k in range(8):
            h = eidx_v[0, pl.ds(off0 + j * CH + k * 16, 16)]
            t = type_v[pl.ds(j * CH + k * 16, 16)]
            idx_v[j, pl.ds(k * 16, 16)] = _flat_idx(h, t)
        pltpu.async_copy(mflat.at[idx_v.at[j]], dp_v.at[j], sem)
        return 0
    lax.fori_loop(0, NFULL, row_body, 0)

    h = eidx_v[0, pl.ds(off0 + NFULL * CH, 16)]
    t = type_v[pl.ds(NFULL * CH, 16)]
    idx_v[NFULL, pl.ds(0, 16)] = _flat_idx(h, t)
    pbase = wid * 256
    for k in range(1, 8):
        idx_v[NFULL, pl.ds(k * 16, 16)] = (pbase + (k * 16) + lane0) << 6
        idx_v[NFULL + 1, pl.ds(k * 16, 16)] = (pbase + (128 + k * 16) + lane0) << 6
    idx_v[NFULL + 1, pl.ds(0, 16)] = (pbase + 128 + lane0) << 6
    pltpu.async_copy(mflat.at[idx_v.at[NFULL]], dp_v.at[NFULL], sem)
    pltpu.async_copy(mflat.at[idx_v.at[NFULL + 1]], dp_v.at[NFULL + 1], sem)

    def drain(j, _):
        pltpu.make_async_copy(mflat.at[idx_v.at[j]], dp_v.at[j], sem).wait()
        return 0
    lax.fori_loop(0, NROW, drain, 0)

    def max_row(j, acc):
        for k in range(8):
            acc = jnp.maximum(acc, dp_v[j, pl.ds(k * 16, 16)])
        return acc
    mvec = lax.fori_loop(0, NFULL, max_row,
                         jnp.full((16,), -jnp.inf, jnp.float32))
    mvec = jnp.maximum(mvec, dp_v[NFULL, pl.ds(0, 16)])

    lane = lax.iota(jnp.int32, 16)

    for sh in (8, 4, 2, 1):
        max_v[...] = mvec
        mvec = jnp.maximum(mvec, plsc.load_gather(max_v, [lane ^ sh]))
    mv = mvec

    def exp_row(j, _):
        for k in range(8):
            v = dp_v[j, pl.ds(k * 16, 16)]
            t = type_v[pl.ds(j * CH + k * 16, 16)]
            plsc.addupdate_scatter(bins, [lane, t], jnp.exp(v - mv))
        return 0
    lax.fori_loop(0, NFULL, exp_row, 0)
    v = dp_v[NFULL, pl.ds(0, 16)]
    t = type_v[pl.ds(NFULL * CH, 16)]
    plsc.addupdate_scatter(bins, [lane, t], jnp.exp(v - mv))

    def merge_body(c, _):
        acc = bins[0, pl.ds(c * 16, 16)]
        for k in range(1, 16):
            acc = acc + bins[k, pl.ds(c * 16, 16)]
        row_v[pl.ds(c * 16, 16)] = acc
        return 0
    lax.fori_loop(0, RPAD // 16, merge_body, 0)

    max_v[...] = mv
    pltpu.sync_copy(row_v, hist_hbm.at[wid])
    pltpu.sync_copy(max_v, maxes_hbm.at[wid])


_sc_edge = pl.kernel(
    _sc_body,
    out_type=(jax.ShapeDtypeStruct((NW, RPAD), jnp.float32),
              jax.ShapeDtypeStruct((NW, 16), jnp.float32)),
    mesh=plsc.VectorSubcoreMesh(core_axis_name="c", subcore_axis_name="s"),
    compiler_params=pltpu.CompilerParams(needs_layout_passes=False),
    scratch_types=[
        pltpu.VMEM((2, NROW * CH), jnp.int32),
        pltpu.VMEM((E_PER,), jnp.int32),
        pltpu.VMEM((NROW, CH), jnp.int32),
        pltpu.VMEM((NROW, CH), jnp.float32),
        pltpu.VMEM((16, RPAD), jnp.float32),
        pltpu.VMEM((RPAD,), jnp.float32),
        pltpu.VMEM((16,), jnp.float32),
        pltpu.SemaphoreType.DMA,
    ],
)


def _fin_body(hist_ref, maxes_ref, rel_ref, o_ref):
    m = jnp.max(maxes_ref[...])
    e = jnp.exp(maxes_ref[:, 0:1] - m)
    t = jnp.sum(hist_ref[...] * e, axis=0)
    z = jnp.sum(t)
    s = t[0:NUM_REL] / z
    o_ref[...] = jnp.maximum(rel_ref[...] * s[:, None], 0.0)


def _finalize(hist, maxes, rel_emb):
    return pl.pallas_call(
        _fin_body,
        out_shape=jax.ShapeDtypeStruct((NUM_REL, HID), jnp.float32),
    )(hist, maxes, rel_emb)


def kernel(x, edge_index, edge_type, rel_emb, W_e2r):
    relp = jnp.pad(rel_emb, ((0, RPAD - NUM_REL), (0, 0)))
    m = _matmul(x, relp, W_e2r)
    hist, maxes = _sc_edge(m.reshape(-1), edge_index, edge_type)
    return _finalize(hist, maxes, rel_emb)


# speedup vs baseline: 46.4666x; 1.0311x over previous
import jax
import jax.numpy as jnp
from jax import lax
from jax.experimental import pallas as pl
from jax.experimental.pallas import tpu as pltpu
from jax.experimental.pallas import tpu_sc as plsc

N_NODES = 10000
N_EDGES = 320000
HID = 128
NUM_REL = 500
RPAD = 512

NC, NS = 2, 16
NW = NC * NS
E_PER = N_EDGES // NW
CH = 128
NROW = 80
NVEC = E_PER // 16
MB = 2000
NPAD = 10240
PLANE = NPAD * HID


def _bf16_split(a):
    hi = a.astype(jnp.bfloat16)
    lo = (a - hi.astype(jnp.float32)).astype(jnp.bfloat16)
    return hi, lo


def _mm_body(x_ref, relp_ref, w_ref, o_ref):
    e_x = lax.dot_general(
        x_ref[...], w_ref[...], (((1,), (1,)), ((), ())),
        preferred_element_type=jnp.float32)
    dims = (((1,), (1,)), ((), ()))
    exh, exl = _bf16_split(e_x)
    relp = relp_ref[...]
    for j in range(4):
        rj = relp[j * HID:(j + 1) * HID, :]
        rh, rl = _bf16_split(rj)
        acc = lax.dot_general(exh, rh, dims, preferred_element_type=jnp.float32)
        acc += lax.dot_general(exh, rl, dims, preferred_element_type=jnp.float32)
        acc += lax.dot_general(exl, rh, dims, preferred_element_type=jnp.float32)
        o_ref[j] = acc


def _matmul(x, rel, w):
    return pl.pallas_call(
        _mm_body,
        grid=(N_NODES // MB,),
        in_specs=[
            pl.BlockSpec((MB, HID), lambda i: (i, 0)),
            pl.BlockSpec((RPAD, HID), lambda i: (0, 0)),
            pl.BlockSpec((HID, HID), lambda i: (0, 0)),
        ],
        out_specs=pl.BlockSpec((4, MB, HID), lambda i: (0, i, 0)),
        out_shape=jax.ShapeDtypeStruct((4, NPAD, HID), jnp.float32),
    )(x, rel, w)


def _sc_body(mflat, eidx_hbm, type_hbm, hist_hbm, maxes_hbm,
             eidx_v, type_v, idx_v, dp_v, bins, row_v, max_v, sem):
    wid = lax.axis_index("s") * NC + lax.axis_index("c")
    base = wid * E_PER
    tile0 = jnp.minimum(base >> 7, jnp.int32(N_EDGES // CH - NROW))
    off0 = base - (tile0 << 7)
    cp1 = pltpu.async_copy(
        eidx_hbm.at[:, pl.ds(pl.multiple_of(tile0 * CH, CH), NROW * CH)],
        eidx_v, sem)
    cp2 = pltpu.async_copy(type_hbm.at[pl.ds(base, E_PER)], type_v, sem)

    zeros16 = jnp.zeros((16,), jnp.float32)
    NFULL = NVEC // 8

    def zero_row(k, _):
        for c in range(RPAD // 16):
            bins[k, pl.ds(c * 16, 16)] = zeros16
        return 0
    lax.fori_loop(0, 16, zero_row, 0)

    cp1.wait()
    cp2.wait()

    lane0 = lax.iota(jnp.int32, 16)

    def _flat_idx(h, t):
        return (t >> 7) * PLANE + (h << 7) + (t & 127)

    def row_body(j, _):
        for k in range(8):
            h = eidx_v[0, pl.ds(off0 + j * CH + k * 16, 16)]
            t = type_v[pl.ds(j * CH + k * 16, 16)]
            idx_v[j, pl.ds(k * 16, 16)] = _flat_idx(h, t)
        pltpu.async_copy(mflat.at[idx_v.at[j]], dp_v.at[j], sem)
        return 0
    lax.fori_loop(0, NFULL, row_body, 0)

    h = eidx_v[0, pl.ds(off0 + NFULL * CH, 16)]
    t = type_v[pl.ds(NFULL * CH, 16)]
    idx_v[NFULL, pl.ds(0, 16)] = _flat_idx(h, t)
    pbase = wid * 256
    for k in range(1, 8):
        idx_v[NFULL, pl.ds(k * 16, 16)] = (pbase + (k * 16) + lane0) << 6
        idx_v[NFULL + 1, pl.ds(k * 16, 16)] = (pbase + (128 + k * 16) + lane0) << 6
    idx_v[NFULL + 1, pl.ds(0, 16)] = (pbase + 128 + lane0) << 6
    pltpu.async_copy(mflat.at[idx_v.at[NFULL]], dp_v.at[NFULL], sem)
    pltpu.async_copy(mflat.at[idx_v.at[NFULL + 1]], dp_v.at[NFULL + 1], sem)

    def drain(j, _):
        pltpu.make_async_copy(mflat.at[idx_v.at[j]], dp_v.at[j], sem).wait()
        return 0
    lax.fori_loop(0, NROW, drain, 0)

    def max_row(j, acc):
        for k in range(8):
            acc = jnp.maximum(acc, dp_v[j, pl.ds(k * 16, 16)])
        return acc
    mvec = lax.fori_loop(0, NFULL, max_row,
                         jnp.full((16,), -jnp.inf, jnp.float32))
    mvec = jnp.maximum(mvec, dp_v[NFULL, pl.ds(0, 16)])

    lane = lax.iota(jnp.int32, 16)

    for sh in (8, 4, 2, 1):
        max_v[...] = mvec
        mvec = jnp.maximum(mvec, plsc.load_gather(max_v, [lane ^ sh]))
    mv = mvec

    def exp_row(j, _):
        for k in range(8):
            v = dp_v[j, pl.ds(k * 16, 16)]
            t = type_v[pl.ds(j * CH + k * 16, 16)]
            plsc.addupdate_scatter(bins, [lane, t], jnp.exp(v - mv))
        return 0
    lax.fori_loop(0, NFULL, exp_row, 0)
    v = dp_v[NFULL, pl.ds(0, 16)]
    t = type_v[pl.ds(NFULL * CH, 16)]
    plsc.addupdate_scatter(bins, [lane, t], jnp.exp(v - mv))

    def merge_body(c, _):
        acc = bins[0, pl.ds(c * 16, 16)]
        for k in range(1, 16):
            acc = acc + bins[k, pl.ds(c * 16, 16)]
        row_v[pl.ds(c * 16, 16)] = acc
        return 0
    lax.fori_loop(0, RPAD // 16, merge_body, 0)

    max_v[...] = mv
    pltpu.sync_copy(row_v, hist_hbm.at[wid])
    pltpu.sync_copy(max_v, maxes_hbm.at[wid])


_sc_edge = pl.kernel(
    _sc_body,
    out_type=(jax.ShapeDtypeStruct((NW, RPAD), jnp.float32),
              jax.ShapeDtypeStruct((NW, 16), jnp.float32)),
    mesh=plsc.VectorSubcoreMesh(core_axis_name="c", subcore_axis_name="s"),
    compiler_params=pltpu.CompilerParams(needs_layout_passes=False),
    scratch_types=[
        pltpu.VMEM((2, NROW * CH), jnp.int32),
        pltpu.VMEM((E_PER,), jnp.int32),
        pltpu.VMEM((NROW, CH), jnp.int32),
        pltpu.VMEM((NROW, CH), jnp.float32),
        pltpu.VMEM((16, RPAD), jnp.float32),
        pltpu.VMEM((RPAD,), jnp.float32),
        pltpu.VMEM((16,), jnp.float32),
        pltpu.SemaphoreType.DMA,
    ],
)


def _fin_body(hist_ref, maxes_ref, rel_ref, o_ref):
    m = jnp.max(maxes_ref[...])
    e = jnp.exp(maxes_ref[:, 0:1] - m)
    t = jnp.sum(hist_ref[...] * e, axis=0)
    z = jnp.sum(t)
    s = t[0:NUM_REL] / z
    o_ref[...] = jnp.maximum(rel_ref[...] * s[:, None], 0.0)


def _finalize(hist, maxes, rel_emb):
    return pl.pallas_call(
        _fin_body,
        out_shape=jax.ShapeDtypeStruct((NUM_REL, HID), jnp.float32),
    )(hist, maxes, rel_emb)


def kernel(x, edge_index, edge_type, rel_emb, W_e2r):
    m = _matmul(x, rel_emb, W_e2r)
    hist, maxes = _sc_edge(m.reshape(-1), edge_index, edge_type)
    return _finalize(hist, maxes, rel_emb)
